# baseline (speedup 1.0000x reference)
import functools

import jax
import jax.numpy as jnp
from jax import lax
from jax.experimental import pallas as pl
from jax.experimental.pallas import tpu as pltpu
from jax.experimental.pallas import tpu_sc as plsc

B, T, D = 2, 4096, 2048
E = 64
TOP_K = 2
TOKENS = B * T
TILE = 512
NTILES = TOKENS // TILE
Q = 4

_NEG_INF = float("-inf")

_info = plsc.get_sparse_core_info()
NC, NS, L = _info.num_cores, _info.num_subcores, _info.num_lanes
NW = NC * NS
TPW = TOKENS // NW
NGROUPS = TPW // L


def _matmul_kernel(x_hbm, w_ref, logits_ref, xbuf, sems):
    i = pl.program_id(0)

    def copy(step, slot):
        return pltpu.make_async_copy(
            x_hbm.at[pl.ds(step * TILE, TILE), :],
            xbuf.at[slot],
            sems.at[slot],
        )

    @pl.when(i == 0)
    def _():
        for q in range(Q):
            copy(q, q).start()

    slot = jax.lax.rem(i, Q)
    copy(i, slot).wait()

    logits_ref[...] = jax.lax.dot_general(
        xbuf[slot], w_ref[...], (((1,), (1,)), ((), ())),
        preferred_element_type=jnp.float32,
    )

    @pl.when(i + Q < NTILES)
    def _():
        copy(i + Q, slot).start()


def _sc_route(logits_hbm, w_hbm, i_hbm, lbuf, wbuf, ibuf, sem):
    wid = lax.axis_index("s") * NC + lax.axis_index("c")
    base = wid * TPW
    pltpu.async_copy(logits_hbm.at[pl.ds(base, TPW), :], lbuf, sem).wait()

    lane = lax.iota(jnp.int32, L)

    def group(g, carry):
        tok = lane + g * L
        zero = jnp.zeros((L,), jnp.int32)
        m1 = plsc.load_gather(lbuf, [tok, zero])
        i1 = zero
        m2 = jnp.full((L,), _NEG_INF, jnp.float32)
        i2 = zero
        for e in range(1, E):
            ev = jnp.full((L,), e, jnp.int32)
            v = plsc.load_gather(lbuf, [tok, ev])
            gt1 = v > m1
            gt2 = v > m2
            m2 = jnp.where(gt1, m1, jnp.where(gt2, v, m2))
            i2 = jnp.where(gt1, i1, jnp.where(gt2, ev, i2))
            m1 = jnp.where(gt1, v, m1)
            i1 = jnp.where(gt1, ev, i1)
        e2 = jnp.exp(m2 - m1)
        denom = 1.0 + e2
        one = jnp.full((L,), 1, jnp.int32)
        plsc.store_scatter(wbuf, [tok, zero], 1.0 / denom)
        plsc.store_scatter(wbuf, [tok, one], e2 / denom)
        plsc.store_scatter(ibuf, [tok, zero], i1)
        plsc.store_scatter(ibuf, [tok, one], i2)
        return carry

    lax.fori_loop(0, NGROUPS, group, 0)

    pltpu.sync_copy(wbuf, w_hbm.at[pl.ds(base, TPW), :])
    pltpu.sync_copy(ibuf, i_hbm.at[pl.ds(base, TPW), :])


_sc_route_call = functools.partial(
    pl.kernel,
    mesh=plsc.VectorSubcoreMesh(core_axis_name="c", subcore_axis_name="s"),
    out_type=[
        jax.ShapeDtypeStruct((TOKENS, TOP_K), jnp.float32),
        jax.ShapeDtypeStruct((TOKENS, TOP_K), jnp.int32),
    ],
    scratch_types=[
        pltpu.VMEM((TPW, E), jnp.float32),
        pltpu.VMEM((TPW, TOP_K), jnp.float32),
        pltpu.VMEM((TPW, TOP_K), jnp.int32),
        pltpu.SemaphoreType.DMA,
    ],
    compiler_params=pltpu.CompilerParams(
        use_tc_tiling_on_sc=False, needs_layout_passes=False
    ),
)(_sc_route)


@jax.jit
def kernel(x, W):
    xt = x.reshape(TOKENS, D)
    logits = pl.pallas_call(
        _matmul_kernel,
        grid=(NTILES,),
        in_specs=[
            pl.BlockSpec(memory_space=pltpu.MemorySpace.HBM),
            pl.BlockSpec((E, D), lambda i: (0, 0)),
        ],
        out_specs=pl.BlockSpec((TILE, E), lambda i: (i, 0)),
        out_shape=jax.ShapeDtypeStruct((TOKENS, E), jnp.float32),
        scratch_shapes=[
            pltpu.VMEM((Q, TILE, D), jnp.float32),
            pltpu.SemaphoreType.DMA((Q,)),
        ],
        compiler_params=pltpu.CompilerParams(
            dimension_semantics=("arbitrary",),
        ),
    )(xt, W)
    weights, indices = _sc_route_call(logits)
    return (
        weights.reshape(B, T, TOP_K),
        indices.reshape(B, T, TOP_K),
        logits.reshape(B, T, E),
    )
